# Initial kernel scaffold; baseline (speedup 1.0000x reference)
import jax
import jax.numpy as jnp
from jax.experimental import pallas as pl


def kernel(continuous, W_continuous, bias):
    raise NotImplementedError("write your pallas kernel here")


# speedup vs baseline: 647.8216x; 647.8216x over previous
import functools

import jax
import jax.numpy as jnp
from jax.experimental import pallas as pl


def _matvec_block(x_ref, w_ref, b_ref, o_ref):
    x = x_ref[...]
    w = w_ref[...]
    d = x.shape[1]
    acc = jnp.sum(x * w, axis=1, keepdims=True)
    o_ref[...] = acc + b_ref[...] * d


@jax.jit
def kernel(continuous, W_continuous, bias):
    n, d = continuous.shape
    out_dim = W_continuous.shape[1]
    w_row = W_continuous.T
    b2 = bias.reshape(1, 1)

    BN = 2048
    grid = (n // BN,)
    out = pl.pallas_call(
        _matvec_block,
        grid=grid,
        in_specs=[
            pl.BlockSpec((BN, d), lambda i: (i, 0)),
            pl.BlockSpec((1, d), lambda i: (0, 0)),
            pl.BlockSpec((1, 1), lambda i: (0, 0)),
        ],
        out_specs=pl.BlockSpec((BN, 1), lambda i: (i, 0)),
        out_shape=jax.ShapeDtypeStruct((n, out_dim), jnp.float32),
    )(continuous, w_row, b2)
    return out
